# baseline (speedup 1.0000x reference)
import functools

import jax
import jax.numpy as jnp
from jax import lax
from jax.experimental import pallas as pl
from jax.experimental.pallas import tpu as pltpu
from jax.experimental.pallas import tpu_sc as plsc

EMB = 64
IMG = 28
BATCH = 16384

_info = plsc.get_sparse_core_info()
_NC = _info.num_cores
_NS = _info.num_subcores
_NW = _NC * _NS
_BPW = BATCH // _NW
_CH = 64
_NCH = _BPW // _CH
_FB = 16

_mesh = plsc.VectorSubcoreMesh(core_axis_name="c", subcore_axis_name="s")


@functools.partial(
    pl.kernel,
    mesh=_mesh,
    out_type=jax.ShapeDtypeStruct((BATCH, EMB), jnp.float32),
    scratch_types=[
        pltpu.VMEM((_NCH, _CH), jnp.int32),
        pltpu.VMEM((_CH, EMB), jnp.float32),
        pltpu.VMEM((_CH, EMB), jnp.float32),
        pltpu.SemaphoreType.DMA,
        pltpu.SemaphoreType.DMA,
    ],
)
def _sc_gather(idx_hbm, table_hbm, out_hbm, idx_v, rows0, rows1, sem0, sem1):
    wid = lax.axis_index("s") * _NC + lax.axis_index("c")
    base = wid * _BPW
    pltpu.sync_copy(idx_hbm.at[wid], idx_v)
    bufs = (rows0, rows1)
    sems = (sem0, sem1)

    def gather_chunk(c, buf, sem):
        for g in range(_CH // _FB):
            vec = idx_v[c, pl.ds(g * _FB, _FB)]
            cps = []
            for k in range(_FB):
                i = vec[k]
                cps.append(
                    pltpu.async_copy(
                        table_hbm.at[pl.ds(i, 1)],
                        buf.at[pl.ds(g * _FB + k, 1)],
                        sem,
                    )
                )
            for cp in cps:
                cp.wait()

    def flush_chunk(c, buf):
        pltpu.sync_copy(buf, out_hbm.at[pl.ds(base + c * _CH, _CH)])

    def step(c, _):
        buf = bufs[c % 2]
        gather_chunk(c, buf, sems[c % 2])
        flush_chunk(c, buf)
        return ()

    for c in range(_NCH):
        step(c, ())


_BM = 1024


def _mm_body(emb_ref, w3_ref, b3_ref, out_ref):
    emb = emb_ref[...]
    for r in range(IMG):
        out_ref[:, r, :] = (
            jnp.dot(emb, w3_ref[r], preferred_element_type=jnp.float32)
            + b3_ref[r]
        )


def kernel(x, table, W, b):
    idx = x.astype(jnp.int32).reshape(_NW, _NCH, _CH)
    emb = _sc_gather(idx, table)
    w3 = W.reshape(EMB, IMG, IMG).transpose(1, 0, 2)
    b3 = b.reshape(IMG, IMG)
    out = pl.pallas_call(
        _mm_body,
        grid=(BATCH // _BM,),
        in_specs=[
            pl.BlockSpec((_BM, EMB), lambda i: (i, 0)),
            pl.BlockSpec((IMG, EMB, IMG), lambda i: (0, 0, 0)),
            pl.BlockSpec((IMG, IMG), lambda i: (0, 0)),
        ],
        out_specs=pl.BlockSpec((_BM, IMG, IMG), lambda i: (i, 0, 0)),
        out_shape=jax.ShapeDtypeStruct((BATCH, IMG, IMG), jnp.float32),
    )(emb, w3, b3)
    return out


# speedup vs baseline: 1.4855x; 1.4855x over previous
import functools

import jax
import jax.numpy as jnp
from jax import lax
from jax.experimental import pallas as pl
from jax.experimental.pallas import tpu as pltpu
from jax.experimental.pallas import tpu_sc as plsc

EMB = 64
IMG = 28
BATCH = 16384

_info = plsc.get_sparse_core_info()
_NC = _info.num_cores
_NS = _info.num_subcores
_NW = _NC * _NS
_BPW = BATCH // _NW
_CH = 64
_NCH = _BPW // _CH
_FB = 16

_mesh = plsc.VectorSubcoreMesh(core_axis_name="c", subcore_axis_name="s")


@functools.partial(
    pl.kernel,
    mesh=_mesh,
    out_type=jax.ShapeDtypeStruct((BATCH, EMB), jnp.float32),
    scratch_types=[
        pltpu.VMEM((_NCH, _CH), jnp.int32),
        pltpu.VMEM((_CH, EMB), jnp.float32),
        pltpu.VMEM((_CH, EMB), jnp.float32),
        pltpu.SemaphoreType.DMA,
        pltpu.SemaphoreType.DMA,
    ],
)
def _sc_gather(idx_hbm, table_hbm, out_hbm, idx_v, rows0, rows1, sem0, sem1):
    wid = lax.axis_index("s") * _NC + lax.axis_index("c")
    base = wid * _BPW
    pltpu.sync_copy(idx_hbm.at[wid], idx_v)
    bufs = (rows0, rows1)
    sems = (sem0, sem1)

    def gather_chunk(c, buf, sem):
        for g in range(_CH // _FB):
            vec = idx_v[c, pl.ds(g * _FB, _FB)]
            cps = []
            for k in range(_FB):
                i = vec[k]
                cps.append(
                    pltpu.async_copy(
                        table_hbm.at[pl.ds(i, 1)],
                        buf.at[pl.ds(g * _FB + k, 1)],
                        sem,
                    )
                )
            for cp in cps:
                cp.wait()

    def flush_chunk(c, buf):
        pltpu.sync_copy(buf, out_hbm.at[pl.ds(base + c * _CH, _CH)])

    def step(c, _):
        buf = bufs[c % 2]
        gather_chunk(c, buf, sems[c % 2])
        flush_chunk(c, buf)
        return ()

    for c in range(_NCH):
        step(c, ())


_BM = 1024


def _mm_body(emb_ref, w_ref, b_ref, out_ref):
    out_ref[...] = (
        jnp.dot(emb_ref[...], w_ref[...], preferred_element_type=jnp.float32)
        + b_ref[...]
    )


def kernel(x, table, W, b):
    idx = x.astype(jnp.int32).reshape(_NW, _NCH, _CH)
    emb = _sc_gather(idx, table)
    out = pl.pallas_call(
        _mm_body,
        grid=(BATCH // _BM,),
        in_specs=[
            pl.BlockSpec((_BM, EMB), lambda i: (i, 0)),
            pl.BlockSpec((EMB, IMG * IMG), lambda i: (0, 0)),
            pl.BlockSpec((1, IMG * IMG), lambda i: (0, 0)),
        ],
        out_specs=pl.BlockSpec((_BM, IMG * IMG), lambda i: (i, 0)),
        out_shape=jax.ShapeDtypeStruct((BATCH, IMG * IMG), jnp.float32),
    )(emb, W, b.reshape(1, IMG * IMG))
    return out.reshape(-1, IMG, IMG)


# speedup vs baseline: 1.6903x; 1.1378x over previous
import functools

import jax
import jax.numpy as jnp
from jax import lax
from jax.experimental import pallas as pl
from jax.experimental.pallas import tpu as pltpu
from jax.experimental.pallas import tpu_sc as plsc

EMB = 64
IMG = 28
BATCH = 16384

_info = plsc.get_sparse_core_info()
_NC = _info.num_cores
_NS = _info.num_subcores
_NW = _NC * _NS
_BPW = BATCH // _NW
_CH = 64
_NCH = _BPW // _CH
_FB = 16

_mesh = plsc.VectorSubcoreMesh(core_axis_name="c", subcore_axis_name="s")


@functools.partial(
    pl.kernel,
    mesh=_mesh,
    out_type=jax.ShapeDtypeStruct((BATCH, EMB), jnp.float32),
    scratch_types=[
        pltpu.VMEM((_NCH, _CH), jnp.int32),
        pltpu.VMEM((_CH, EMB), jnp.float32),
        pltpu.VMEM((_CH, EMB), jnp.float32),
        pltpu.SemaphoreType.DMA,
        pltpu.SemaphoreType.DMA,
    ],
)
def _sc_gather(idx_hbm, table_hbm, out_hbm, idx_v, rows0, rows1, sem0, sem1):
    wid = lax.axis_index("s") * _NC + lax.axis_index("c")
    base = wid * _BPW
    pltpu.sync_copy(idx_hbm.at[wid], idx_v)
    bufs = (rows0, rows1)
    sems = (sem0, sem1)

    def gather_chunk(c, buf, sem):
        for g in range(_CH // _FB):
            vec = idx_v[c, pl.ds(g * _FB, _FB)]
            cps = []
            for k in range(_FB):
                i = vec[k]
                cps.append(
                    pltpu.async_copy(
                        table_hbm.at[pl.ds(i, 1)],
                        buf.at[pl.ds(g * _FB + k, 1)],
                        sem,
                    )
                )
            for cp in cps:
                cp.wait()

    for c in range(_NCH):
        buf = bufs[c % 2]
        gather_chunk(c, buf, sems[c % 2])
        pltpu.sync_copy(buf, out_hbm.at[pl.ds(base + c * _CH, _CH)])


_BN = 2048


def _mm_body(w_ref, emb_ref, b_ref, out_ref):
    out_ref[...] = (
        lax.dot_general(
            w_ref[...],
            emb_ref[...],
            ((( 0,), (1,)), ((), ())),
            preferred_element_type=jnp.float32,
        )
        + b_ref[...]
    )


def kernel(x, table, W, b):
    idx = x.astype(jnp.int32).reshape(_NW, _NCH, _CH)
    emb = _sc_gather(idx, table)
    outT = pl.pallas_call(
        _mm_body,
        grid=(BATCH // _BN,),
        in_specs=[
            pl.BlockSpec((EMB, IMG * IMG), lambda i: (0, 0)),
            pl.BlockSpec((_BN, EMB), lambda i: (i, 0)),
            pl.BlockSpec((IMG * IMG, 1), lambda i: (0, 0)),
        ],
        out_specs=pl.BlockSpec((IMG * IMG, _BN), lambda i: (0, i)),
        out_shape=jax.ShapeDtypeStruct((IMG * IMG, BATCH), jnp.float32),
    )(W, emb, b.reshape(IMG * IMG, 1))
    return outT.T.reshape(BATCH, IMG, IMG)
